# baseline (speedup 1.0000x reference)
import math

import jax
import jax.numpy as jnp
import numpy as np
from jax.experimental import pallas as pl
from jax.experimental.pallas import tpu as pltpu

S = 2048
D_MODEL = 2048
N_HEADS = 16
HEAD_DIM = D_MODEL // N_HEADS
CLIP_QKV = 8.0
ALIBI_BIAS_MAX = 8

M_TILE = 256
QB = 256
KB = 256
SCALE = HEAD_DIM ** -0.5
NEG = -1e30


def _alibi_slopes_np(total_num_heads, alibi_bias_max):
    next_pow2 = 2 ** math.ceil(math.log2(total_num_heads))
    m = np.arange(1, next_pow2 + 1, dtype=np.float32) * (alibi_bias_max / next_pow2)
    slopes = 1.0 / np.power(2.0, m)
    if next_pow2 != total_num_heads:
        slopes = np.concatenate([slopes[1::2], slopes[::2]])[:total_num_heads]
    return slopes.astype(np.float32)


def _qkv_body(h_ref, w_ref, lnw_ref, lnb_ref, o_ref):
    j = pl.program_id(0)
    x = jax.lax.dot_general(
        h_ref[...], w_ref[...], (((1,), (0,)), ((), ())),
        preferred_element_type=jnp.float32)
    x = jnp.clip(x, -CLIP_QKV, CLIP_QKV)

    @pl.when(j < 2)
    def _():
        mu = jnp.mean(x, axis=-1, keepdims=True)
        var = jnp.mean(x * x, axis=-1, keepdims=True) - mu * mu
        ln = (x - mu) * jax.lax.rsqrt(var + 1e-5) * lnw_ref[0] + lnb_ref[0]
        o_ref[...] = ln.astype(jnp.bfloat16)

    @pl.when(j == 2)
    def _():
        o_ref[...] = x.astype(jnp.bfloat16)


def _attn_body(slopes_ref, q_ref, k_ref, v_ref, wo_ref, o_ref):
    qb = pl.program_id(0)
    nkb = qb + 1
    tri = (jax.lax.broadcasted_iota(jnp.int32, (QB, KB), 1)
           <= jax.lax.broadcasted_iota(jnp.int32, (QB, KB), 0))
    jcol = jax.lax.broadcasted_iota(jnp.int32, (1, KB), 1).astype(jnp.float32)

    for h in range(N_HEADS):
        slope = slopes_ref[h]
        qh = q_ref[:, h * HEAD_DIM:(h + 1) * HEAD_DIM]

        def chunk(kb, carry):
            m, l, acc = carry
            kc = k_ref[pl.ds(kb * KB, KB), h * HEAD_DIM:(h + 1) * HEAD_DIM]
            vc = v_ref[pl.ds(kb * KB, KB), h * HEAD_DIM:(h + 1) * HEAD_DIM]
            s = jax.lax.dot_general(
                qh, kc, (((1,), (1,)), ((), ())),
                preferred_element_type=jnp.float32) * SCALE
            cb = slope * (jcol + (kb * KB).astype(jnp.float32))
            s = s + cb
            s = jnp.where(jnp.logical_or(kb < qb, tri), s, NEG)
            m_new = jnp.maximum(m, jnp.max(s, axis=-1, keepdims=True))
            alpha = jnp.exp(m - m_new)
            p = jnp.exp(s - m_new)
            l = l * alpha + jnp.sum(p, axis=-1, keepdims=True)
            acc = acc * alpha + jax.lax.dot_general(
                p.astype(jnp.bfloat16), vc, (((1,), (0,)), ((), ())),
                preferred_element_type=jnp.float32)
            return m_new, l, acc

        m0 = jnp.full((QB, 1), NEG, jnp.float32)
        l0 = jnp.zeros((QB, 1), jnp.float32)
        acc0 = jnp.zeros((QB, HEAD_DIM), jnp.float32)
        _, l, acc = jax.lax.fori_loop(0, nkb, chunk, (m0, l0, acc0))
        ctx = acc / l
        contrib = jax.lax.dot_general(
            ctx.astype(jnp.bfloat16), wo_ref[h * HEAD_DIM:(h + 1) * HEAD_DIM, :],
            (((1,), (0,)), ((), ())),
            preferred_element_type=jnp.float32)
        if h == 0:
            o_ref[...] = contrib
        else:
            o_ref[...] += contrib


def kernel(position_ids, hidden_states, layernums, KV_cache, Wqkv_w,
           q_ln_w, q_ln_b, k_ln_w, k_ln_b, out_w):
    del position_ids, layernums, KV_cache
    hs = hidden_states.reshape(S, D_MODEL).astype(jnp.bfloat16)
    w_qkv = Wqkv_w.astype(jnp.bfloat16)
    w_out = out_w.astype(jnp.bfloat16)
    ln_w = jnp.stack([q_ln_w, k_ln_w, jnp.ones_like(q_ln_w)]).reshape(3, 1, D_MODEL)
    ln_b = jnp.stack([q_ln_b, k_ln_b, jnp.zeros_like(q_ln_b)]).reshape(3, 1, D_MODEL)

    qkv = pl.pallas_call(
        _qkv_body,
        grid=(3, S // M_TILE),
        in_specs=[
            pl.BlockSpec((M_TILE, D_MODEL), lambda j, i: (i, 0)),
            pl.BlockSpec((D_MODEL, D_MODEL), lambda j, i: (0, j)),
            pl.BlockSpec((1, 1, D_MODEL), lambda j, i: (j, 0, 0)),
            pl.BlockSpec((1, 1, D_MODEL), lambda j, i: (j, 0, 0)),
        ],
        out_specs=pl.BlockSpec((M_TILE, D_MODEL), lambda j, i: (i, j)),
        out_shape=jax.ShapeDtypeStruct((S, 3 * D_MODEL), jnp.bfloat16),
    )(hs, w_qkv, ln_w, ln_b)

    slopes = jnp.asarray(_alibi_slopes_np(N_HEADS, ALIBI_BIAS_MAX))

    out = pl.pallas_call(
        _attn_body,
        grid=(S // QB,),
        in_specs=[
            pl.BlockSpec(memory_space=pltpu.SMEM),
            pl.BlockSpec((QB, D_MODEL), lambda i: (i, 0)),
            pl.BlockSpec((S, D_MODEL), lambda i: (0, 1)),
            pl.BlockSpec((S, D_MODEL), lambda i: (0, 2)),
            pl.BlockSpec((D_MODEL, D_MODEL), lambda i: (0, 0)),
        ],
        out_specs=pl.BlockSpec((QB, D_MODEL), lambda i: (i, 0)),
        out_shape=jax.ShapeDtypeStruct((S, D_MODEL), jnp.float32),
    )(slopes, qkv, qkv, qkv, w_out)

    return out.reshape(1, S, D_MODEL)


# speedup vs baseline: 2.0575x; 2.0575x over previous
import math

import jax
import jax.numpy as jnp
import numpy as np
from jax.experimental import pallas as pl
from jax.experimental.pallas import tpu as pltpu

S = 2048
D_MODEL = 2048
N_HEADS = 16
HEAD_DIM = D_MODEL // N_HEADS
CLIP_QKV = 8.0
ALIBI_BIAS_MAX = 8

M_TILE = 256
QB = 256
KB = 256
O_TILE = 512
SCALE = HEAD_DIM ** -0.5
NEG = -1e30


def _alibi_slopes_np(total_num_heads, alibi_bias_max):
    next_pow2 = 2 ** math.ceil(math.log2(total_num_heads))
    m = np.arange(1, next_pow2 + 1, dtype=np.float32) * (alibi_bias_max / next_pow2)
    slopes = 1.0 / np.power(2.0, m)
    if next_pow2 != total_num_heads:
        slopes = np.concatenate([slopes[1::2], slopes[::2]])[:total_num_heads]
    return slopes.astype(np.float32)


def _qkv_body(h_ref, w_ref, lnw_ref, lnb_ref, o_ref):
    j = pl.program_id(0)
    x = jax.lax.dot_general(
        h_ref[...], w_ref[...], (((1,), (0,)), ((), ())),
        preferred_element_type=jnp.float32)
    x = jnp.clip(x, -CLIP_QKV, CLIP_QKV)

    @pl.when(j < 2)
    def _():
        mu = jnp.mean(x, axis=-1, keepdims=True)
        var = jnp.mean(x * x, axis=-1, keepdims=True) - mu * mu
        ln = (x - mu) * jax.lax.rsqrt(var + 1e-5) * lnw_ref[0] + lnb_ref[0]
        o_ref[...] = ln.astype(jnp.bfloat16)

    @pl.when(j == 2)
    def _():
        o_ref[...] = x.astype(jnp.bfloat16)


def _attn_body(slopes_ref, q_ref, k_ref, v_ref, ctx_ref, s_scr):
    h = pl.program_id(0)
    slope = slopes_ref[h]
    tri = (jax.lax.broadcasted_iota(jnp.int32, (QB, KB), 1)
           <= jax.lax.broadcasted_iota(jnp.int32, (QB, KB), 0))
    jcol = jax.lax.broadcasted_iota(jnp.int32, (1, KB), 1).astype(jnp.float32)
    cbase = slope * jcol

    for qb in range(S // QB):
        qh = q_ref[qb * QB:(qb + 1) * QB, :]
        w = (qb + 1) * KB
        m = jnp.full((QB, 1), NEG, jnp.float32)
        for kb in range(qb + 1):
            kc = k_ref[kb * KB:(kb + 1) * KB, :]
            s = jax.lax.dot_general(
                qh, kc, (((1,), (1,)), ((), ())),
                preferred_element_type=jnp.float32)
            s = s + (cbase + slope * (kb * KB))
            if kb == qb:
                s = jnp.where(tri, s, NEG)
            s_scr[:, kb * KB:(kb + 1) * KB] = s
            m = jnp.maximum(m, jnp.max(s, axis=-1, keepdims=True))
        p = jnp.exp(s_scr[:, :w] - m)
        l = jnp.sum(p, axis=-1, keepdims=True)
        ctx = jax.lax.dot_general(
            p.astype(jnp.bfloat16), v_ref[:w, :], (((1,), (0,)), ((), ())),
            preferred_element_type=jnp.float32) / l
        ctx_ref[qb * QB:(qb + 1) * QB, :] = ctx.astype(jnp.bfloat16)


def _proj_body(x_ref, w_ref, o_ref):
    o_ref[...] = jax.lax.dot_general(
        x_ref[...], w_ref[...], (((1,), (0,)), ((), ())),
        preferred_element_type=jnp.float32)


def kernel(position_ids, hidden_states, layernums, KV_cache, Wqkv_w,
           q_ln_w, q_ln_b, k_ln_w, k_ln_b, out_w):
    del position_ids, layernums, KV_cache
    hs = hidden_states.reshape(S, D_MODEL).astype(jnp.bfloat16)
    w_qkv = Wqkv_w.astype(jnp.bfloat16)
    w_out = out_w.astype(jnp.bfloat16)
    ln_w = jnp.stack([q_ln_w * SCALE, k_ln_w, jnp.ones_like(q_ln_w)]).reshape(3, 1, D_MODEL)
    ln_b = jnp.stack([q_ln_b * SCALE, k_ln_b, jnp.zeros_like(q_ln_b)]).reshape(3, 1, D_MODEL)

    qkv = pl.pallas_call(
        _qkv_body,
        grid=(3, S // M_TILE),
        in_specs=[
            pl.BlockSpec((M_TILE, D_MODEL), lambda j, i: (i, 0)),
            pl.BlockSpec((D_MODEL, D_MODEL), lambda j, i: (0, j)),
            pl.BlockSpec((1, 1, D_MODEL), lambda j, i: (j, 0, 0)),
            pl.BlockSpec((1, 1, D_MODEL), lambda j, i: (j, 0, 0)),
        ],
        out_specs=pl.BlockSpec((M_TILE, D_MODEL), lambda j, i: (i, j)),
        out_shape=jax.ShapeDtypeStruct((S, 3 * D_MODEL), jnp.bfloat16),
    )(hs, w_qkv, ln_w, ln_b)

    slopes = jnp.asarray(_alibi_slopes_np(N_HEADS, ALIBI_BIAS_MAX))

    ctx = pl.pallas_call(
        _attn_body,
        grid=(N_HEADS,),
        in_specs=[
            pl.BlockSpec(memory_space=pltpu.SMEM),
            pl.BlockSpec((S, HEAD_DIM), lambda h: (0, h)),
            pl.BlockSpec((S, HEAD_DIM), lambda h: (0, N_HEADS + h)),
            pl.BlockSpec((S, HEAD_DIM), lambda h: (0, 2 * N_HEADS + h)),
        ],
        out_specs=pl.BlockSpec((S, HEAD_DIM), lambda h: (0, h)),
        out_shape=jax.ShapeDtypeStruct((S, D_MODEL), jnp.bfloat16),
        scratch_shapes=[pltpu.VMEM((QB, S), jnp.float32)],
    )(slopes, qkv, qkv, qkv)

    out = pl.pallas_call(
        _proj_body,
        grid=(S // O_TILE,),
        in_specs=[
            pl.BlockSpec((O_TILE, D_MODEL), lambda i: (i, 0)),
            pl.BlockSpec((D_MODEL, D_MODEL), lambda i: (0, 0)),
        ],
        out_specs=pl.BlockSpec((O_TILE, D_MODEL), lambda i: (i, 0)),
        out_shape=jax.ShapeDtypeStruct((S, D_MODEL), jnp.float32),
    )(ctx, w_out)

    return out.reshape(1, S, D_MODEL)


# speedup vs baseline: 2.1311x; 1.0358x over previous
import math

import jax
import jax.numpy as jnp
import numpy as np
from jax.experimental import pallas as pl
from jax.experimental.pallas import tpu as pltpu

S = 2048
D_MODEL = 2048
N_HEADS = 16
HEAD_DIM = D_MODEL // N_HEADS
CLIP_QKV = 8.0
ALIBI_BIAS_MAX = 8

M_TILE = 512
QB = 256
KB = 256
O_TILE = 256
LOG2E = 1.4426950408889634
SCALE = HEAD_DIM ** -0.5 * LOG2E
NEG = -1e30


def _alibi_slopes_np(total_num_heads, alibi_bias_max):
    next_pow2 = 2 ** math.ceil(math.log2(total_num_heads))
    m = np.arange(1, next_pow2 + 1, dtype=np.float32) * (alibi_bias_max / next_pow2)
    slopes = 1.0 / np.power(2.0, m)
    if next_pow2 != total_num_heads:
        slopes = np.concatenate([slopes[1::2], slopes[::2]])[:total_num_heads]
    return slopes.astype(np.float32)


def _qkv_body(h_ref, w_ref, lnw_ref, lnb_ref, o_ref):
    j = pl.program_id(0)
    x = jax.lax.dot_general(
        h_ref[...], w_ref[...], (((1,), (0,)), ((), ())),
        preferred_element_type=jnp.float32)
    x = jnp.clip(x, -CLIP_QKV, CLIP_QKV)

    @pl.when(j < 2)
    def _():
        mu = jnp.mean(x, axis=-1, keepdims=True)
        var = jnp.mean(x * x, axis=-1, keepdims=True) - mu * mu
        ln = (x - mu) * jax.lax.rsqrt(var + 1e-5) * lnw_ref[0] + lnb_ref[0]
        o_ref[...] = ln.astype(jnp.bfloat16)

    @pl.when(j == 2)
    def _():
        o_ref[...] = x.astype(jnp.bfloat16)


def _attn_body(slopes_ref, q_ref, k_ref, v_ref, ctx_ref, s_scr):
    h = pl.program_id(0)
    slope = slopes_ref[h] * LOG2E
    tri = (jax.lax.broadcasted_iota(jnp.int32, (QB, KB), 1)
           <= jax.lax.broadcasted_iota(jnp.int32, (QB, KB), 0))
    jcol = jax.lax.broadcasted_iota(jnp.int32, (1, KB), 1).astype(jnp.float32)
    cbase = slope * jcol

    for qb in range(S // QB):
        qh = q_ref[qb * QB:(qb + 1) * QB, :]
        w = (qb + 1) * KB
        m = jnp.full((QB, 1), NEG, jnp.float32)
        for kb in range(qb + 1):
            kc = k_ref[kb * KB:(kb + 1) * KB, :]
            s = jax.lax.dot_general(
                qh, kc, (((1,), (1,)), ((), ())),
                preferred_element_type=jnp.float32)
            s = s + (cbase + slope * (kb * KB))
            if kb == qb:
                s = jnp.where(tri, s, NEG)
            s_scr[:, kb * KB:(kb + 1) * KB] = s
            m = jnp.maximum(m, jnp.max(s, axis=-1, keepdims=True))
        p = jnp.exp2(s_scr[:, :w] - m)
        l = jnp.sum(p, axis=-1, keepdims=True)
        ctx = jax.lax.dot_general(
            p.astype(jnp.bfloat16), v_ref[:w, :], (((1,), (0,)), ((), ())),
            preferred_element_type=jnp.float32) / l
        ctx_ref[qb * QB:(qb + 1) * QB, :] = ctx.astype(jnp.bfloat16)


def _proj_body(x_ref, w_ref, o_ref):
    o_ref[...] = jax.lax.dot_general(
        x_ref[...], w_ref[...], (((1,), (0,)), ((), ())),
        preferred_element_type=jnp.float32)


def kernel(position_ids, hidden_states, layernums, KV_cache, Wqkv_w,
           q_ln_w, q_ln_b, k_ln_w, k_ln_b, out_w):
    del position_ids, layernums, KV_cache
    hs = hidden_states.reshape(S, D_MODEL).astype(jnp.bfloat16)
    w_qkv = Wqkv_w.astype(jnp.bfloat16)
    w_out = out_w.astype(jnp.bfloat16)
    ln_w = jnp.stack([q_ln_w * SCALE, k_ln_w, jnp.ones_like(q_ln_w)]).reshape(3, 1, D_MODEL)
    ln_b = jnp.stack([q_ln_b * SCALE, k_ln_b, jnp.zeros_like(q_ln_b)]).reshape(3, 1, D_MODEL)

    qkv = pl.pallas_call(
        _qkv_body,
        grid=(3, S // M_TILE),
        in_specs=[
            pl.BlockSpec((M_TILE, D_MODEL), lambda j, i: (i, 0)),
            pl.BlockSpec((D_MODEL, D_MODEL), lambda j, i: (0, j)),
            pl.BlockSpec((1, 1, D_MODEL), lambda j, i: (j, 0, 0)),
            pl.BlockSpec((1, 1, D_MODEL), lambda j, i: (j, 0, 0)),
        ],
        out_specs=pl.BlockSpec((M_TILE, D_MODEL), lambda j, i: (i, j)),
        out_shape=jax.ShapeDtypeStruct((S, 3 * D_MODEL), jnp.bfloat16),
    )(hs, w_qkv, ln_w, ln_b)

    slopes = jnp.asarray(_alibi_slopes_np(N_HEADS, ALIBI_BIAS_MAX))

    ctx = pl.pallas_call(
        _attn_body,
        grid=(N_HEADS,),
        in_specs=[
            pl.BlockSpec(memory_space=pltpu.SMEM),
            pl.BlockSpec((S, HEAD_DIM), lambda h: (0, h)),
            pl.BlockSpec((S, HEAD_DIM), lambda h: (0, N_HEADS + h)),
            pl.BlockSpec((S, HEAD_DIM), lambda h: (0, 2 * N_HEADS + h)),
        ],
        out_specs=pl.BlockSpec((S, HEAD_DIM), lambda h: (0, h)),
        out_shape=jax.ShapeDtypeStruct((S, D_MODEL), jnp.bfloat16),
        scratch_shapes=[pltpu.VMEM((QB, S), jnp.float32)],
    )(slopes, qkv, qkv, qkv)

    out = pl.pallas_call(
        _proj_body,
        grid=(S // O_TILE,),
        in_specs=[
            pl.BlockSpec((O_TILE, D_MODEL), lambda i: (i, 0)),
            pl.BlockSpec((D_MODEL, D_MODEL), lambda i: (0, 0)),
        ],
        out_specs=pl.BlockSpec((O_TILE, D_MODEL), lambda i: (i, 0)),
        out_shape=jax.ShapeDtypeStruct((S, D_MODEL), jnp.float32),
    )(ctx, w_out)

    return out.reshape(1, S, D_MODEL)


# speedup vs baseline: 2.4170x; 1.1341x over previous
import math

import jax
import jax.numpy as jnp
import numpy as np
from jax.experimental import pallas as pl
from jax.experimental.pallas import tpu as pltpu

S = 2048
D_MODEL = 2048
N_HEADS = 16
HEAD_DIM = D_MODEL // N_HEADS
CLIP_QKV = 8.0
ALIBI_BIAS_MAX = 8

M_TILE = 512
QB = 256
KB = 256
O_TILE = 256
LOG2E = 1.4426950408889634
SCALE = HEAD_DIM ** -0.5 * LOG2E
NEG = -1e30


def _alibi_slopes_np(total_num_heads, alibi_bias_max):
    next_pow2 = 2 ** math.ceil(math.log2(total_num_heads))
    m = np.arange(1, next_pow2 + 1, dtype=np.float32) * (alibi_bias_max / next_pow2)
    slopes = 1.0 / np.power(2.0, m)
    if next_pow2 != total_num_heads:
        slopes = np.concatenate([slopes[1::2], slopes[::2]])[:total_num_heads]
    return slopes.astype(np.float32)


def _qkv_body(h_ref, w_ref, lnw_ref, lnb_ref, o_ref, wbf_scr):
    j = pl.program_id(0)

    @pl.when(pl.program_id(1) == 0)
    def _():
        wbf_scr[...] = w_ref[...].astype(jnp.bfloat16)

    x = jax.lax.dot_general(
        h_ref[...].astype(jnp.bfloat16), wbf_scr[...], (((1,), (0,)), ((), ())),
        preferred_element_type=jnp.float32)
    x = jnp.clip(x, -CLIP_QKV, CLIP_QKV)

    @pl.when(j < 2)
    def _():
        mu = jnp.mean(x, axis=-1, keepdims=True)
        var = jnp.mean(x * x, axis=-1, keepdims=True) - mu * mu
        ln = (x - mu) * jax.lax.rsqrt(var + 1e-5) * lnw_ref[0] + lnb_ref[0]
        o_ref[...] = ln.astype(jnp.bfloat16)

    @pl.when(j == 2)
    def _():
        o_ref[...] = x.astype(jnp.bfloat16)


def _attn_body(slopes_ref, q_ref, k_ref, v_ref, ctx_ref, s_scr):
    h = pl.program_id(0)
    slope = slopes_ref[h] * LOG2E
    tri = (jax.lax.broadcasted_iota(jnp.int32, (QB, KB), 1)
           <= jax.lax.broadcasted_iota(jnp.int32, (QB, KB), 0))
    jcol = jax.lax.broadcasted_iota(jnp.int32, (1, KB), 1).astype(jnp.float32)
    cbase = slope * jcol

    for qb in range(S // QB):
        qh = q_ref[qb * QB:(qb + 1) * QB, :]
        w = (qb + 1) * KB
        m = jnp.full((QB, 1), NEG, jnp.float32)
        for kb in range(qb + 1):
            kc = k_ref[kb * KB:(kb + 1) * KB, :]
            s = jax.lax.dot_general(
                qh, kc, (((1,), (1,)), ((), ())),
                preferred_element_type=jnp.float32)
            s = s + (cbase + slope * (kb * KB))
            if kb == qb:
                s = jnp.where(tri, s, NEG)
            s_scr[:, kb * KB:(kb + 1) * KB] = s
            m = jnp.maximum(m, jnp.max(s, axis=-1, keepdims=True))
        p = jnp.exp2(s_scr[:, :w] - m)
        l = jnp.sum(p, axis=-1, keepdims=True)
        ctx = jax.lax.dot_general(
            p.astype(jnp.bfloat16), v_ref[:w, :], (((1,), (0,)), ((), ())),
            preferred_element_type=jnp.float32) / l
        ctx_ref[qb * QB:(qb + 1) * QB, :] = ctx.astype(jnp.bfloat16)


def _proj_body(x_ref, w_ref, o_ref, wbf_scr):
    @pl.when(pl.program_id(0) == 0)
    def _():
        wbf_scr[...] = w_ref[...].astype(jnp.bfloat16)

    o_ref[...] = jax.lax.dot_general(
        x_ref[...], wbf_scr[...], (((1,), (0,)), ((), ())),
        preferred_element_type=jnp.float32)


def kernel(position_ids, hidden_states, layernums, KV_cache, Wqkv_w,
           q_ln_w, q_ln_b, k_ln_w, k_ln_b, out_w):
    del position_ids, layernums, KV_cache
    hs = hidden_states.reshape(S, D_MODEL)
    ln_w = jnp.stack([q_ln_w * SCALE, k_ln_w, jnp.ones_like(q_ln_w)]).reshape(3, 1, D_MODEL)
    ln_b = jnp.stack([q_ln_b * SCALE, k_ln_b, jnp.zeros_like(q_ln_b)]).reshape(3, 1, D_MODEL)

    qkv = pl.pallas_call(
        _qkv_body,
        grid=(3, S // M_TILE),
        in_specs=[
            pl.BlockSpec((M_TILE, D_MODEL), lambda j, i: (i, 0)),
            pl.BlockSpec((D_MODEL, D_MODEL), lambda j, i: (0, j)),
            pl.BlockSpec((1, 1, D_MODEL), lambda j, i: (j, 0, 0)),
            pl.BlockSpec((1, 1, D_MODEL), lambda j, i: (j, 0, 0)),
        ],
        out_specs=pl.BlockSpec((M_TILE, D_MODEL), lambda j, i: (i, j)),
        out_shape=jax.ShapeDtypeStruct((S, 3 * D_MODEL), jnp.bfloat16),
        scratch_shapes=[pltpu.VMEM((D_MODEL, D_MODEL), jnp.bfloat16)],
    )(hs, Wqkv_w, ln_w, ln_b)

    slopes = jnp.asarray(_alibi_slopes_np(N_HEADS, ALIBI_BIAS_MAX))

    ctx = pl.pallas_call(
        _attn_body,
        grid=(N_HEADS,),
        in_specs=[
            pl.BlockSpec(memory_space=pltpu.SMEM),
            pl.BlockSpec((S, HEAD_DIM), lambda h: (0, h)),
            pl.BlockSpec((S, HEAD_DIM), lambda h: (0, N_HEADS + h)),
            pl.BlockSpec((S, HEAD_DIM), lambda h: (0, 2 * N_HEADS + h)),
        ],
        out_specs=pl.BlockSpec((S, HEAD_DIM), lambda h: (0, h)),
        out_shape=jax.ShapeDtypeStruct((S, D_MODEL), jnp.bfloat16),
        scratch_shapes=[pltpu.VMEM((QB, S), jnp.float32)],
    )(slopes, qkv, qkv, qkv)

    out = pl.pallas_call(
        _proj_body,
        grid=(S // O_TILE,),
        in_specs=[
            pl.BlockSpec((O_TILE, D_MODEL), lambda i: (i, 0)),
            pl.BlockSpec((D_MODEL, D_MODEL), lambda i: (0, 0)),
        ],
        out_specs=pl.BlockSpec((O_TILE, D_MODEL), lambda i: (i, 0)),
        out_shape=jax.ShapeDtypeStruct((S, D_MODEL), jnp.float32),
        scratch_shapes=[pltpu.VMEM((D_MODEL, D_MODEL), jnp.bfloat16)],
    )(ctx, out_w)

    return out.reshape(1, S, D_MODEL)


# speedup vs baseline: 2.4240x; 1.0029x over previous
import math

import jax
import jax.numpy as jnp
import numpy as np
from jax.experimental import pallas as pl
from jax.experimental.pallas import tpu as pltpu

S = 2048
D_MODEL = 2048
N_HEADS = 16
HEAD_DIM = D_MODEL // N_HEADS
CLIP_QKV = 8.0
ALIBI_BIAS_MAX = 8

M_TILE = 512
QB = 512
KB = 512
O_TILE = 256
LOG2E = 1.4426950408889634
SCALE = HEAD_DIM ** -0.5 * LOG2E
NEG = -1e30


def _alibi_slopes_np(total_num_heads, alibi_bias_max):
    next_pow2 = 2 ** math.ceil(math.log2(total_num_heads))
    m = np.arange(1, next_pow2 + 1, dtype=np.float32) * (alibi_bias_max / next_pow2)
    slopes = 1.0 / np.power(2.0, m)
    if next_pow2 != total_num_heads:
        slopes = np.concatenate([slopes[1::2], slopes[::2]])[:total_num_heads]
    return slopes.astype(np.float32)


def _qkv_body(h_ref, w_ref, lnw_ref, lnb_ref, o_ref, wbf_scr):
    j = pl.program_id(0)

    @pl.when(pl.program_id(1) == 0)
    def _():
        wbf_scr[...] = w_ref[...].astype(jnp.bfloat16)

    x = jax.lax.dot_general(
        h_ref[...].astype(jnp.bfloat16), wbf_scr[...], (((1,), (0,)), ((), ())),
        preferred_element_type=jnp.float32)
    x = jnp.clip(x, -CLIP_QKV, CLIP_QKV)

    @pl.when(j < 2)
    def _():
        mu = jnp.mean(x, axis=-1, keepdims=True)
        var = jnp.mean(x * x, axis=-1, keepdims=True) - mu * mu
        ln = (x - mu) * jax.lax.rsqrt(var + 1e-5) * lnw_ref[0] + lnb_ref[0]
        o_ref[...] = ln.astype(jnp.bfloat16)

    @pl.when(j == 2)
    def _():
        o_ref[...] = x.astype(jnp.bfloat16)


def _attn_body(slopes_ref, q_ref, k_ref, v_ref, ctx_ref, s_scr):
    h = pl.program_id(0)
    slope = slopes_ref[h] * LOG2E
    tri = (jax.lax.broadcasted_iota(jnp.int32, (QB, KB), 1)
           <= jax.lax.broadcasted_iota(jnp.int32, (QB, KB), 0))
    jcol = jax.lax.broadcasted_iota(jnp.int32, (1, KB), 1).astype(jnp.float32)
    cbase = slope * jcol

    for qb in range(S // QB):
        qh = q_ref[qb * QB:(qb + 1) * QB, :]
        w = (qb + 1) * KB
        m = jnp.full((QB, 1), NEG, jnp.float32)
        for kb in range(qb + 1):
            kc = k_ref[kb * KB:(kb + 1) * KB, :]
            s = jax.lax.dot_general(
                qh, kc, (((1,), (1,)), ((), ())),
                preferred_element_type=jnp.float32)
            s = s + (cbase + slope * (kb * KB))
            if kb == qb:
                s = jnp.where(tri, s, NEG)
            s_scr[:, kb * KB:(kb + 1) * KB] = s
            m = jnp.maximum(m, jnp.max(s, axis=-1, keepdims=True))
        p = jnp.exp2(s_scr[:, :w] - m)
        l = jnp.sum(p, axis=-1, keepdims=True)
        ctx = jax.lax.dot_general(
            p.astype(jnp.bfloat16), v_ref[:w, :], (((1,), (0,)), ((), ())),
            preferred_element_type=jnp.float32) / l
        ctx_ref[qb * QB:(qb + 1) * QB, :] = ctx.astype(jnp.bfloat16)


def _proj_body(x_ref, w_ref, o_ref, wbf_scr):
    @pl.when(pl.program_id(0) == 0)
    def _():
        wbf_scr[...] = w_ref[...].astype(jnp.bfloat16)

    o_ref[...] = jax.lax.dot_general(
        x_ref[...], wbf_scr[...], (((1,), (0,)), ((), ())),
        preferred_element_type=jnp.float32)


def kernel(position_ids, hidden_states, layernums, KV_cache, Wqkv_w,
           q_ln_w, q_ln_b, k_ln_w, k_ln_b, out_w):
    del position_ids, layernums, KV_cache
    hs = hidden_states.reshape(S, D_MODEL)
    ln_w = jnp.stack([q_ln_w * SCALE, k_ln_w, jnp.ones_like(q_ln_w)]).reshape(3, 1, D_MODEL)
    ln_b = jnp.stack([q_ln_b * SCALE, k_ln_b, jnp.zeros_like(q_ln_b)]).reshape(3, 1, D_MODEL)

    qkv = pl.pallas_call(
        _qkv_body,
        grid=(3, S // M_TILE),
        in_specs=[
            pl.BlockSpec((M_TILE, D_MODEL), lambda j, i: (i, 0)),
            pl.BlockSpec((D_MODEL, D_MODEL), lambda j, i: (0, j)),
            pl.BlockSpec((1, 1, D_MODEL), lambda j, i: (j, 0, 0)),
            pl.BlockSpec((1, 1, D_MODEL), lambda j, i: (j, 0, 0)),
        ],
        out_specs=pl.BlockSpec((M_TILE, D_MODEL), lambda j, i: (i, j)),
        out_shape=jax.ShapeDtypeStruct((S, 3 * D_MODEL), jnp.bfloat16),
        scratch_shapes=[pltpu.VMEM((D_MODEL, D_MODEL), jnp.bfloat16)],
    )(hs, Wqkv_w, ln_w, ln_b)

    slopes = jnp.asarray(_alibi_slopes_np(N_HEADS, ALIBI_BIAS_MAX))

    ctx = pl.pallas_call(
        _attn_body,
        grid=(N_HEADS,),
        in_specs=[
            pl.BlockSpec(memory_space=pltpu.SMEM),
            pl.BlockSpec((S, HEAD_DIM), lambda h: (0, h)),
            pl.BlockSpec((S, HEAD_DIM), lambda h: (0, N_HEADS + h)),
            pl.BlockSpec((S, HEAD_DIM), lambda h: (0, 2 * N_HEADS + h)),
        ],
        out_specs=pl.BlockSpec((S, HEAD_DIM), lambda h: (0, h)),
        out_shape=jax.ShapeDtypeStruct((S, D_MODEL), jnp.bfloat16),
        scratch_shapes=[pltpu.VMEM((QB, S), jnp.float32)],
    )(slopes, qkv, qkv, qkv)

    out = pl.pallas_call(
        _proj_body,
        grid=(S // O_TILE,),
        in_specs=[
            pl.BlockSpec((O_TILE, D_MODEL), lambda i: (i, 0)),
            pl.BlockSpec((D_MODEL, D_MODEL), lambda i: (0, 0)),
        ],
        out_specs=pl.BlockSpec((O_TILE, D_MODEL), lambda i: (i, 0)),
        out_shape=jax.ShapeDtypeStruct((S, D_MODEL), jnp.float32),
        scratch_shapes=[pltpu.VMEM((D_MODEL, D_MODEL), jnp.bfloat16)],
    )(ctx, out_w)

    return out.reshape(1, S, D_MODEL)


# speedup vs baseline: 2.4594x; 1.0146x over previous
import math

import jax
import jax.numpy as jnp
import numpy as np
from jax.experimental import pallas as pl
from jax.experimental.pallas import tpu as pltpu

S = 2048
D_MODEL = 2048
N_HEADS = 16
HEAD_DIM = D_MODEL // N_HEADS
CLIP_QKV = 8.0
ALIBI_BIAS_MAX = 8

M_TILE = 512
QB = 512
KB = 512
O_TILE = 256
LOG2E = 1.4426950408889634
SCALE = HEAD_DIM ** -0.5 * LOG2E
NEG = -1e30


def _alibi_slopes_np(total_num_heads, alibi_bias_max):
    next_pow2 = 2 ** math.ceil(math.log2(total_num_heads))
    m = np.arange(1, next_pow2 + 1, dtype=np.float32) * (alibi_bias_max / next_pow2)
    slopes = 1.0 / np.power(2.0, m)
    if next_pow2 != total_num_heads:
        slopes = np.concatenate([slopes[1::2], slopes[::2]])[:total_num_heads]
    return slopes.astype(np.float32)


def _qkv_body(h_ref, w_ref, lnw_ref, lnb_ref, o_ref, wbf_scr):
    j = pl.program_id(0)

    @pl.when(pl.program_id(1) == 0)
    def _():
        wbf_scr[...] = w_ref[...].astype(jnp.bfloat16)

    x = jax.lax.dot_general(
        h_ref[...].astype(jnp.bfloat16), wbf_scr[...], (((1,), (0,)), ((), ())),
        preferred_element_type=jnp.float32)
    x = jnp.clip(x, -CLIP_QKV, CLIP_QKV)

    @pl.when(j < 2)
    def _():
        mu = jnp.mean(x, axis=-1, keepdims=True)
        var = jnp.mean(x * x, axis=-1, keepdims=True) - mu * mu
        ln = (x - mu) * jax.lax.rsqrt(var + 1e-5) * lnw_ref[0] + lnb_ref[0]
        o_ref[...] = ln.astype(jnp.bfloat16)

    @pl.when(j == 2)
    def _():
        o_ref[...] = x.astype(jnp.bfloat16)


def _attn_body(slopes_ref, q_ref, k_ref, v_ref, ctx_ref, s_scr, vaug_scr):
    h = pl.program_id(0)
    slope = slopes_ref[h] * LOG2E
    tri = (jax.lax.broadcasted_iota(jnp.int32, (QB, KB), 1)
           <= jax.lax.broadcasted_iota(jnp.int32, (QB, KB), 0))
    jcol = jax.lax.broadcasted_iota(jnp.int32, (1, KB), 1).astype(jnp.float32)
    cbase = slope * jcol

    @pl.when(h == 0)
    def _():
        vaug_scr[:, HEAD_DIM:] = jnp.ones((S, HEAD_DIM), jnp.bfloat16)

    vaug_scr[:, :HEAD_DIM] = v_ref[...]

    for qb in range(S // QB):
        qh = q_ref[qb * QB:(qb + 1) * QB, :]
        w = (qb + 1) * KB
        m = jnp.full((QB, 1), NEG, jnp.float32)
        for kb in range(qb + 1):
            kc = k_ref[kb * KB:(kb + 1) * KB, :]
            s = jax.lax.dot_general(
                qh, kc, (((1,), (1,)), ((), ())),
                preferred_element_type=jnp.float32)
            s = s + (cbase + slope * (kb * KB))
            if kb == qb:
                s = jnp.where(tri, s, NEG)
            s_scr[:, kb * KB:(kb + 1) * KB] = s
            m = jnp.maximum(m, jnp.max(s, axis=-1, keepdims=True))
        p = jnp.exp2(s_scr[:, :w] - m)
        cl = jax.lax.dot_general(
            p.astype(jnp.bfloat16), vaug_scr[:w, :], (((1,), (0,)), ((), ())),
            preferred_element_type=jnp.float32)
        ctx = cl[:, :HEAD_DIM] / cl[:, HEAD_DIM:HEAD_DIM + 1]
        ctx_ref[qb * QB:(qb + 1) * QB, :] = ctx.astype(jnp.bfloat16)


def _proj_body(x_ref, w_ref, o_ref, wbf_scr):
    @pl.when(pl.program_id(0) == 0)
    def _():
        wbf_scr[...] = w_ref[...].astype(jnp.bfloat16)

    o_ref[...] = jax.lax.dot_general(
        x_ref[...], wbf_scr[...], (((1,), (0,)), ((), ())),
        preferred_element_type=jnp.float32)


def kernel(position_ids, hidden_states, layernums, KV_cache, Wqkv_w,
           q_ln_w, q_ln_b, k_ln_w, k_ln_b, out_w):
    del position_ids, layernums, KV_cache
    hs = hidden_states.reshape(S, D_MODEL)
    ln_w = jnp.stack([q_ln_w * SCALE, k_ln_w, jnp.ones_like(q_ln_w)]).reshape(3, 1, D_MODEL)
    ln_b = jnp.stack([q_ln_b * SCALE, k_ln_b, jnp.zeros_like(q_ln_b)]).reshape(3, 1, D_MODEL)

    qkv = pl.pallas_call(
        _qkv_body,
        grid=(3, S // M_TILE),
        in_specs=[
            pl.BlockSpec((M_TILE, D_MODEL), lambda j, i: (i, 0)),
            pl.BlockSpec((D_MODEL, D_MODEL), lambda j, i: (0, j)),
            pl.BlockSpec((1, 1, D_MODEL), lambda j, i: (j, 0, 0)),
            pl.BlockSpec((1, 1, D_MODEL), lambda j, i: (j, 0, 0)),
        ],
        out_specs=pl.BlockSpec((M_TILE, D_MODEL), lambda j, i: (i, j)),
        out_shape=jax.ShapeDtypeStruct((S, 3 * D_MODEL), jnp.bfloat16),
        scratch_shapes=[pltpu.VMEM((D_MODEL, D_MODEL), jnp.bfloat16)],
    )(hs, Wqkv_w, ln_w, ln_b)

    slopes = jnp.asarray(_alibi_slopes_np(N_HEADS, ALIBI_BIAS_MAX))

    ctx = pl.pallas_call(
        _attn_body,
        grid=(N_HEADS,),
        in_specs=[
            pl.BlockSpec(memory_space=pltpu.SMEM),
            pl.BlockSpec((S, HEAD_DIM), lambda h: (0, h)),
            pl.BlockSpec((S, HEAD_DIM), lambda h: (0, N_HEADS + h)),
            pl.BlockSpec((S, HEAD_DIM), lambda h: (0, 2 * N_HEADS + h)),
        ],
        out_specs=pl.BlockSpec((S, HEAD_DIM), lambda h: (0, h)),
        out_shape=jax.ShapeDtypeStruct((S, D_MODEL), jnp.bfloat16),
        scratch_shapes=[pltpu.VMEM((QB, S), jnp.float32),
                        pltpu.VMEM((S, 2 * HEAD_DIM), jnp.bfloat16)],
    )(slopes, qkv, qkv, qkv)

    out = pl.pallas_call(
        _proj_body,
        grid=(S // O_TILE,),
        in_specs=[
            pl.BlockSpec((O_TILE, D_MODEL), lambda i: (i, 0)),
            pl.BlockSpec((D_MODEL, D_MODEL), lambda i: (0, 0)),
        ],
        out_specs=pl.BlockSpec((O_TILE, D_MODEL), lambda i: (i, 0)),
        out_shape=jax.ShapeDtypeStruct((S, D_MODEL), jnp.float32),
        scratch_shapes=[pltpu.VMEM((D_MODEL, D_MODEL), jnp.bfloat16)],
    )(ctx, out_w)

    return out.reshape(1, S, D_MODEL)


# speedup vs baseline: 2.4620x; 1.0011x over previous
import math

import jax
import jax.numpy as jnp
import numpy as np
from jax.experimental import pallas as pl
from jax.experimental.pallas import tpu as pltpu

S = 2048
D_MODEL = 2048
N_HEADS = 16
HEAD_DIM = D_MODEL // N_HEADS
CLIP_QKV = 8.0
ALIBI_BIAS_MAX = 8

M_TILE = 512
QB = 512
KB = 512
O_TILE = 512
LOG2E = 1.4426950408889634
SCALE = HEAD_DIM ** -0.5 * LOG2E
NEG = -1e30


def _alibi_slopes_np(total_num_heads, alibi_bias_max):
    next_pow2 = 2 ** math.ceil(math.log2(total_num_heads))
    m = np.arange(1, next_pow2 + 1, dtype=np.float32) * (alibi_bias_max / next_pow2)
    slopes = 1.0 / np.power(2.0, m)
    if next_pow2 != total_num_heads:
        slopes = np.concatenate([slopes[1::2], slopes[::2]])[:total_num_heads]
    return slopes.astype(np.float32)


def _qkv_body(h_ref, w_ref, lnw_ref, lnb_ref, o_ref, wbf_scr):
    j = pl.program_id(0)

    @pl.when(pl.program_id(1) == 0)
    def _():
        wbf_scr[...] = w_ref[...].astype(jnp.bfloat16)

    x = jax.lax.dot_general(
        h_ref[...].astype(jnp.bfloat16), wbf_scr[...], (((1,), (0,)), ((), ())),
        preferred_element_type=jnp.float32)
    x = jnp.clip(x, -CLIP_QKV, CLIP_QKV)

    @pl.when(j < 2)
    def _():
        mu = jnp.mean(x, axis=-1, keepdims=True)
        var = jnp.mean(x * x, axis=-1, keepdims=True) - mu * mu
        ln = (x - mu) * jax.lax.rsqrt(var + 1e-5) * lnw_ref[0] + lnb_ref[0]
        o_ref[...] = ln.astype(jnp.bfloat16)

    @pl.when(j == 2)
    def _():
        o_ref[...] = x.astype(jnp.bfloat16)


def _attn_body(slopes_ref, q_ref, k_ref, v_ref, ctx_ref, s_scr, vaug_scr):
    h = pl.program_id(0)
    slope = slopes_ref[h] * LOG2E
    tri = (jax.lax.broadcasted_iota(jnp.int32, (QB, KB), 1)
           <= jax.lax.broadcasted_iota(jnp.int32, (QB, KB), 0))
    jcol = jax.lax.broadcasted_iota(jnp.int32, (1, KB), 1).astype(jnp.float32)
    cbase = slope * jcol

    @pl.when(h == 0)
    def _():
        vaug_scr[:, HEAD_DIM:] = jnp.ones((S, HEAD_DIM), jnp.bfloat16)

    vaug_scr[:, :HEAD_DIM] = v_ref[...]

    for qb in range(S // QB):
        qh = q_ref[qb * QB:(qb + 1) * QB, :]
        w = (qb + 1) * KB
        m = jnp.full((QB, 1), NEG, jnp.float32)
        for kb in range(qb + 1):
            kc = k_ref[kb * KB:(kb + 1) * KB, :]
            s = jax.lax.dot_general(
                qh, kc, (((1,), (1,)), ((), ())),
                preferred_element_type=jnp.float32)
            s = s + (cbase + slope * (kb * KB))
            if kb == qb:
                s = jnp.where(tri, s, NEG)
            s_scr[:, kb * KB:(kb + 1) * KB] = s
            m = jnp.maximum(m, jnp.max(s, axis=-1, keepdims=True))
        p = jnp.exp2(s_scr[:, :w] - m)
        cl = jax.lax.dot_general(
            p.astype(jnp.bfloat16), vaug_scr[:w, :], (((1,), (0,)), ((), ())),
            preferred_element_type=jnp.float32)
        ctx = cl[:, :HEAD_DIM] / cl[:, HEAD_DIM:HEAD_DIM + 1]
        ctx_ref[qb * QB:(qb + 1) * QB, :] = ctx.astype(jnp.bfloat16)


def _proj_body(x_ref, w_ref, o_ref, wbf_scr):
    @pl.when(pl.program_id(0) == 0)
    def _():
        wbf_scr[...] = w_ref[...].astype(jnp.bfloat16)

    o_ref[...] = jax.lax.dot_general(
        x_ref[...], wbf_scr[...], (((1,), (0,)), ((), ())),
        preferred_element_type=jnp.float32)


def kernel(position_ids, hidden_states, layernums, KV_cache, Wqkv_w,
           q_ln_w, q_ln_b, k_ln_w, k_ln_b, out_w):
    del position_ids, layernums, KV_cache
    hs = hidden_states.reshape(S, D_MODEL)
    ln_w = jnp.stack([q_ln_w * SCALE, k_ln_w, jnp.ones_like(q_ln_w)]).reshape(3, 1, D_MODEL)
    ln_b = jnp.stack([q_ln_b * SCALE, k_ln_b, jnp.zeros_like(q_ln_b)]).reshape(3, 1, D_MODEL)

    qkv = pl.pallas_call(
        _qkv_body,
        grid=(3, S // M_TILE),
        in_specs=[
            pl.BlockSpec((M_TILE, D_MODEL), lambda j, i: (i, 0)),
            pl.BlockSpec((D_MODEL, D_MODEL), lambda j, i: (0, j)),
            pl.BlockSpec((1, 1, D_MODEL), lambda j, i: (j, 0, 0)),
            pl.BlockSpec((1, 1, D_MODEL), lambda j, i: (j, 0, 0)),
        ],
        out_specs=pl.BlockSpec((M_TILE, D_MODEL), lambda j, i: (i, j)),
        out_shape=jax.ShapeDtypeStruct((S, 3 * D_MODEL), jnp.bfloat16),
        scratch_shapes=[pltpu.VMEM((D_MODEL, D_MODEL), jnp.bfloat16)],
    )(hs, Wqkv_w, ln_w, ln_b)

    slopes = jnp.asarray(_alibi_slopes_np(N_HEADS, ALIBI_BIAS_MAX))

    ctx = pl.pallas_call(
        _attn_body,
        grid=(N_HEADS,),
        in_specs=[
            pl.BlockSpec(memory_space=pltpu.SMEM),
            pl.BlockSpec((S, HEAD_DIM), lambda h: (0, h)),
            pl.BlockSpec((S, HEAD_DIM), lambda h: (0, N_HEADS + h)),
            pl.BlockSpec((S, HEAD_DIM), lambda h: (0, 2 * N_HEADS + h)),
        ],
        out_specs=pl.BlockSpec((S, HEAD_DIM), lambda h: (0, h)),
        out_shape=jax.ShapeDtypeStruct((S, D_MODEL), jnp.bfloat16),
        scratch_shapes=[pltpu.VMEM((QB, S), jnp.float32),
                        pltpu.VMEM((S, 2 * HEAD_DIM), jnp.bfloat16)],
    )(slopes, qkv, qkv, qkv)

    out = pl.pallas_call(
        _proj_body,
        grid=(S // O_TILE,),
        in_specs=[
            pl.BlockSpec((O_TILE, D_MODEL), lambda i: (i, 0)),
            pl.BlockSpec((D_MODEL, D_MODEL), lambda i: (0, 0)),
        ],
        out_specs=pl.BlockSpec((O_TILE, D_MODEL), lambda i: (i, 0)),
        out_shape=jax.ShapeDtypeStruct((S, D_MODEL), jnp.float32),
        scratch_shapes=[pltpu.VMEM((D_MODEL, D_MODEL), jnp.bfloat16)],
    )(ctx, out_w)

    return out.reshape(1, S, D_MODEL)
